# baseline (speedup 1.0000x reference)
import functools

import jax
import jax.numpy as jnp
from jax import lax
from jax.experimental import pallas as pl
from jax.experimental.pallas import tpu as pltpu
from jax.experimental.pallas import tpu_sc as plsc

EMB_DIM = 64
BPC = 4


@functools.cache
def _make_gather(batch: int, n_fields: int, n_emb: int):
  NC, NS = 2, 16
  NW = NC * NS
  chunk = BPC * n_fields
  assert chunk <= 128
  n_chunks = batch // BPC
  assert batch % (BPC * NW) == 0
  ch_per_w = n_chunks // NW
  assert ch_per_w % 4 == 0

  mesh = plsc.VectorSubcoreMesh(core_axis_name="c", subcore_axis_name="s")

  @functools.partial(
      pl.kernel,
      out_type=jax.ShapeDtypeStruct(
          (n_chunks, BPC * n_fields, EMB_DIM), jnp.bfloat16
      ),
      mesh=mesh,
      scratch_types=[
          pltpu.VMEM((ch_per_w, chunk), jnp.int32),
          pltpu.VMEM((4, chunk, EMB_DIM), jnp.bfloat16),
          pltpu.SemaphoreType.DMA((4,)),
          pltpu.SemaphoreType.DMA((4,)),
      ],
      compiler_params=pltpu.CompilerParams(use_tc_tiling_on_sc=False),
  )
  def grab(idx_hbm, table_hbm, out_hbm, idx_v, rows_v, gsem, osem):
    wid = lax.axis_index("s") * NC + lax.axis_index("c")
    base_chunk = wid * ch_per_w
    pltpu.sync_copy(idx_hbm.at[pl.ds(base_chunk, ch_per_w)], idx_v)

    def gather(c, p):
      return pltpu.make_async_copy(
          table_hbm.at[idx_v.at[c]], rows_v.at[p], gsem.at[p]
      )

    def store(c, p):
      return pltpu.make_async_copy(
          rows_v.at[p], out_hbm.at[base_chunk + c], osem.at[p]
      )

    gather(0, 0).start()
    gather(1, 1).start()

    @pl.loop(0, ch_per_w, step=4)
    def _(c0):
      for p in range(4):
        c = c0 + p
        gather(c, p).wait()

        @pl.when(c >= 2)
        def _():
          store(c - 2, (p + 2) % 4).wait()

        store(c, p).start()

        @pl.when(c + 2 < ch_per_w)
        def _():
          gather(c + 2, (p + 2) % 4).start()

    store(ch_per_w - 2, (ch_per_w - 2) % 4).wait()
    store(ch_per_w - 1, (ch_per_w - 1) % 4).wait()

  return grab


def kernel(input, embedding_weight):
  b, f = input.shape
  idx = input.astype(jnp.int32).reshape(b // BPC, BPC * f)
  wbf = embedding_weight.astype(jnp.bfloat16)
  grab = _make_gather(b, f, embedding_weight.shape[0])
  return grab(idx, wbf).reshape(b, f, EMB_DIM)
